# baseline (speedup 1.0000x reference)
import jax
import jax.numpy as jnp
from jax.experimental import pallas as pl
from jax.experimental.pallas import tpu as pltpu

_CHUNK = 256
_DEPTH = 24


def _router_pipeline(x_hbm, w_ref, logits_hbm, mask_hbm, *refs):
    xbufs = refs[:_DEPTH]
    lbuf, mbuf, insem, outsem = refs[_DEPTH:]
    n_chunks = x_hbm.shape[0] // _CHUNK
    half_rows = (n_chunks // 2) * _CHUNK

    def in_copy(c):
        return pltpu.make_async_copy(
            x_hbm.at[pl.ds(c * _CHUNK, _CHUNK), :],
            xbufs[c % _DEPTH],
            insem.at[c % _DEPTH],
        )

    def out_copy(h):
        rows = pl.ds(h * half_rows, half_rows)
        return (
            pltpu.make_async_copy(
                lbuf.at[rows, :], logits_hbm.at[rows, :], outsem.at[2 * h]),
            pltpu.make_async_copy(
                mbuf.at[rows, :], mask_hbm.at[rows, :], outsem.at[2 * h + 1]),
        )

    for c in range(_DEPTH):
        in_copy(c).start()

    for c in range(n_chunks):
        in_copy(c).wait()
        logits = jax.lax.dot_general(
            xbufs[c % _DEPTH][...],
            w_ref[...],
            dimension_numbers=(((1,), (1,)), ((), ())),
            preferred_element_type=jnp.float32,
        )
        lbuf[pl.ds(c * _CHUNK, _CHUNK), :] = logits
        mbuf[pl.ds(c * _CHUNK, _CHUNK), :] = (logits > 0).astype(jnp.int8)
        if c + _DEPTH < n_chunks:
            in_copy(c + _DEPTH).start()
        if c == n_chunks // 2 - 1:
            for cp in out_copy(0):
                cp.start()

    for cp in out_copy(1):
        cp.start()
    for h in range(2):
        for cp in out_copy(h):
            cp.wait()


def kernel(x, weight):
    m, k = x.shape
    e = weight.shape[0]
    logits, mask = pl.pallas_call(
        _router_pipeline,
        in_specs=[
            pl.BlockSpec(memory_space=pl.ANY),
            pl.BlockSpec(memory_space=pltpu.VMEM),
        ],
        out_specs=[
            pl.BlockSpec(memory_space=pl.ANY),
            pl.BlockSpec(memory_space=pl.ANY),
        ],
        out_shape=[
            jax.ShapeDtypeStruct((m, e), jnp.float32),
            jax.ShapeDtypeStruct((m, e), jnp.int8),
        ],
        scratch_shapes=(
            [pltpu.VMEM((_CHUNK, k), jnp.float32) for _ in range(_DEPTH)]
            + [
                pltpu.VMEM((m, e), jnp.float32),
                pltpu.VMEM((m, e), jnp.int8),
                pltpu.SemaphoreType.DMA((_DEPTH,)),
                pltpu.SemaphoreType.DMA((4,)),
            ]
        ),
    )(x, weight)
    return (logits, mask.astype(jnp.bool_))


# speedup vs baseline: 1.0675x; 1.0675x over previous
import jax
import jax.numpy as jnp
from jax.experimental import pallas as pl
from jax.experimental.pallas import tpu as pltpu

_CHUNK = 512
_DEPTH = 8
_OD = 2


def _router_pipeline(x_hbm, w_ref, logits_hbm, mask_hbm,
                     xbuf, lbuf, mbuf, insem, lsem, msem):
    n_chunks = x_hbm.shape[0] // _CHUNK

    def in_copy(c, slot):
        return pltpu.make_async_copy(
            x_hbm.at[pl.ds(c * _CHUNK, _CHUNK), :], xbuf.at[slot], insem.at[slot])

    for j in range(_DEPTH):
        in_copy(j, j).start()

    def body(i, _):
        slot = jax.lax.rem(i, _DEPTH)
        oslot = jax.lax.rem(i, _OD)
        in_copy(i, slot).wait()

        @pl.when(i >= _OD)
        def _():
            pltpu.make_async_copy(
                lbuf.at[oslot],
                logits_hbm.at[pl.ds((i - _OD) * _CHUNK, _CHUNK), :],
                lsem.at[oslot]).wait()
            pltpu.make_async_copy(
                mbuf.at[oslot],
                mask_hbm.at[pl.ds((i - _OD) * _CHUNK, _CHUNK), :],
                msem.at[oslot]).wait()

        logits = jax.lax.dot_general(
            xbuf[slot],
            w_ref[...],
            dimension_numbers=(((1,), (1,)), ((), ())),
            preferred_element_type=jnp.float32,
        )
        lbuf[oslot] = logits
        mbuf[oslot] = (logits > 0).astype(jnp.int8)

        pltpu.make_async_copy(
            lbuf.at[oslot],
            logits_hbm.at[pl.ds(i * _CHUNK, _CHUNK), :],
            lsem.at[oslot]).start()
        pltpu.make_async_copy(
            mbuf.at[oslot],
            mask_hbm.at[pl.ds(i * _CHUNK, _CHUNK), :],
            msem.at[oslot]).start()

        @pl.when(i + _DEPTH < n_chunks)
        def _():
            in_copy(i + _DEPTH, slot).start()

        return 0

    jax.lax.fori_loop(0, n_chunks, body, 0)

    for t in range(_OD):
        c = n_chunks - _OD + t
        oslot = c % _OD
        pltpu.make_async_copy(
            lbuf.at[oslot],
            logits_hbm.at[pl.ds(c * _CHUNK, _CHUNK), :],
            lsem.at[oslot]).wait()
        pltpu.make_async_copy(
            mbuf.at[oslot],
            mask_hbm.at[pl.ds(c * _CHUNK, _CHUNK), :],
            msem.at[oslot]).wait()


def kernel(x, weight):
    m, k = x.shape
    e = weight.shape[0]
    logits, mask = pl.pallas_call(
        _router_pipeline,
        in_specs=[
            pl.BlockSpec(memory_space=pl.ANY),
            pl.BlockSpec(memory_space=pltpu.VMEM),
        ],
        out_specs=[
            pl.BlockSpec(memory_space=pl.ANY),
            pl.BlockSpec(memory_space=pl.ANY),
        ],
        out_shape=[
            jax.ShapeDtypeStruct((m, e), jnp.float32),
            jax.ShapeDtypeStruct((m, e), jnp.int8),
        ],
        scratch_shapes=[
            pltpu.VMEM((_DEPTH, _CHUNK, k), jnp.float32),
            pltpu.VMEM((_OD, _CHUNK, e), jnp.float32),
            pltpu.VMEM((_OD, _CHUNK, e), jnp.int8),
            pltpu.SemaphoreType.DMA((_DEPTH,)),
            pltpu.SemaphoreType.DMA((_OD,)),
            pltpu.SemaphoreType.DMA((_OD,)),
        ],
    )(x, weight)
    return (logits, mask.astype(jnp.bool_))


# speedup vs baseline: 1.0909x; 1.0220x over previous
import jax
import jax.numpy as jnp
from jax.experimental import pallas as pl
from jax.experimental.pallas import tpu as pltpu

_CHUNK = 256
_DEPTH = 8
_OD = 2


def _router_pipeline(x_hbm, w_ref, logits_hbm, mask_hbm,
                     xbuf, lbuf, mbuf, insem, lsem, msem):
    n_chunks = x_hbm.shape[0] // _CHUNK

    def in_copy(c, slot):
        return pltpu.make_async_copy(
            x_hbm.at[pl.ds(c * _CHUNK, _CHUNK), :], xbuf.at[slot], insem.at[slot])

    for j in range(_DEPTH):
        in_copy(j, j).start()

    def body(i, _):
        slot = jax.lax.rem(i, _DEPTH)
        oslot = jax.lax.rem(i, _OD)
        in_copy(i, slot).wait()

        @pl.when(i >= _OD)
        def _():
            pltpu.make_async_copy(
                lbuf.at[oslot],
                logits_hbm.at[pl.ds((i - _OD) * _CHUNK, _CHUNK), :],
                lsem.at[oslot]).wait()
            pltpu.make_async_copy(
                mbuf.at[oslot],
                mask_hbm.at[pl.ds((i - _OD) * _CHUNK, _CHUNK), :],
                msem.at[oslot]).wait()

        logits = jax.lax.dot_general(
            xbuf[slot],
            w_ref[...],
            dimension_numbers=(((1,), (1,)), ((), ())),
            preferred_element_type=jnp.float32,
        )
        lbuf[oslot] = logits
        mbuf[oslot] = (logits > 0).astype(jnp.int8)

        pltpu.make_async_copy(
            lbuf.at[oslot],
            logits_hbm.at[pl.ds(i * _CHUNK, _CHUNK), :],
            lsem.at[oslot]).start()
        pltpu.make_async_copy(
            mbuf.at[oslot],
            mask_hbm.at[pl.ds(i * _CHUNK, _CHUNK), :],
            msem.at[oslot]).start()

        @pl.when(i + _DEPTH < n_chunks)
        def _():
            in_copy(i + _DEPTH, slot).start()

        return 0

    jax.lax.fori_loop(0, n_chunks, body, 0)

    for t in range(_OD):
        c = n_chunks - _OD + t
        oslot = c % _OD
        pltpu.make_async_copy(
            lbuf.at[oslot],
            logits_hbm.at[pl.ds(c * _CHUNK, _CHUNK), :],
            lsem.at[oslot]).wait()
        pltpu.make_async_copy(
            mbuf.at[oslot],
            mask_hbm.at[pl.ds(c * _CHUNK, _CHUNK), :],
            msem.at[oslot]).wait()


def kernel(x, weight):
    m, k = x.shape
    e = weight.shape[0]
    logits, mask = pl.pallas_call(
        _router_pipeline,
        in_specs=[
            pl.BlockSpec(memory_space=pl.ANY),
            pl.BlockSpec(memory_space=pltpu.VMEM),
        ],
        out_specs=[
            pl.BlockSpec(memory_space=pl.ANY),
            pl.BlockSpec(memory_space=pl.ANY),
        ],
        out_shape=[
            jax.ShapeDtypeStruct((m, e), jnp.float32),
            jax.ShapeDtypeStruct((m, e), jnp.int8),
        ],
        scratch_shapes=[
            pltpu.VMEM((_DEPTH, _CHUNK, k), jnp.float32),
            pltpu.VMEM((_OD, _CHUNK, e), jnp.float32),
            pltpu.VMEM((_OD, _CHUNK, e), jnp.int8),
            pltpu.SemaphoreType.DMA((_DEPTH,)),
            pltpu.SemaphoreType.DMA((_OD,)),
            pltpu.SemaphoreType.DMA((_OD,)),
        ],
    )(x, weight)
    return (logits, mask.astype(jnp.bool_))
